# baseline (speedup 1.0000x reference)
import functools

import jax
import jax.numpy as jnp
from jax import lax
from jax.experimental import pallas as pl
from jax.experimental.pallas import tpu as pltpu
from jax.experimental.pallas import tpu_sc as plsc

N = 50000
NPAD = 50176
DUMMY = N
E = 800000
EPAD = 819200
NW = 32
EW = EPAD // NW
K = 128
NB = EW // K
RPS = NPAD // 16
CW = 32
RB = 256
NRB = NPAD // RB

_mesh = plsc.VectorSubcoreMesh(
    core_axis_name="c", subcore_axis_name="s", num_cores=2, num_subcores=16)
_sc_params = pltpu.CompilerParams(use_tc_tiling_on_sc=False)


def _zero_scratch(zb, rows, width):
    z16 = jnp.zeros((16,), jnp.float32)

    def zrow(r, _):
        for j in range(width // 16):
            zb[r, pl.ds(16 * j, 16)] = z16
        return 0

    lax.fori_loop(0, rows, zrow, 0)


SB = 8
NSB = NB // SB
GR = 2
NG = NB // GR
NG0 = 180
NG1 = 20
ZR = 196


def _sc_deg_body(src_hbm, dst_hbm, deg_out, gi_hbm, sib, gib, dib,
                 ones_v, zb, acc):
    cid = lax.axis_index("c")
    sid = lax.axis_index("s")
    wid = sid * 2 + cid
    rbase = wid * (EW // K)

    o16 = jnp.ones((16,), jnp.float32)

    def orow(r, _):
        ones_v[r, :] = o16
        return 0

    lax.fori_loop(0, K, orow, 0)
    _zero_scratch(zb, 64, 16)

    def z(i, _):
        pltpu.sync_copy(zb, acc.at[pl.ds(sid * RPS + i * 64, 64)])
        return 0

    lax.fori_loop(0, RPS // 64, z, 0)
    plsc.subcore_barrier()

    def step(b, _):
        rb = rbase + b * SB
        pltpu.sync_copy(src_hbm.at[pl.ds(rb, SB), :], sib)
        pltpu.sync_copy(dst_hbm.at[pl.ds(rb, SB), :], dib)
        for k in range(SB):
            for j in range(K // 16):
                s = sib[k, pl.ds(16 * j, 16)]
                d = dib[k, pl.ds(16 * j, 16)]
                gib[k, pl.ds(16 * j, 16)] = jnp.where(s == d, DUMMY, s)
        for k in range(SB):
            pltpu.sync_copy(ones_v, acc.at[gib.at[k]], add=True)
        pltpu.sync_copy(gib, gi_hbm.at[pl.ds(rb, SB), :])
        return 0

    lax.fori_loop(0, NSB, step, 0)
    plsc.subcore_barrier()
    pltpu.sync_copy(acc.at[pl.ds(sid * RPS, RPS)],
                    deg_out.at[pl.ds(cid * NPAD + sid * RPS, RPS)])


_sc_deg = pl.kernel(
    _sc_deg_body,
    out_type=(jax.ShapeDtypeStruct((2 * NPAD, 16), jnp.float32),
              jax.ShapeDtypeStruct((EPAD // K, K), jnp.int32)),
    mesh=_mesh,
    scratch_types=[
        pltpu.VMEM((SB, K), jnp.int32),
        pltpu.VMEM((SB, K), jnp.int32),
        pltpu.VMEM((SB, K), jnp.int32),
        pltpu.VMEM((K, 16), jnp.float32),
        pltpu.VMEM((64, 16), jnp.float32),
        pltpu.VMEM_SHARED((NPAD, 16), jnp.float32),
    ],
    compiler_params=_sc_params,
)


def _make_sc_spmm(C):
    def body(*refs):
        gi_hbm, dst_hbm = refs[0], refs[1]
        tables = refs[2:2 + C]
        outs = refs[2 + C:2 + 2 * C]
        (gi3, di3, rows0, rows1, zb, acc,
         gsem0, gsem1, ssem0, ssem1) = refs[2 + 2 * C:2 + 2 * C + 10]
        igs = refs[2 + 2 * C + 10:2 + 2 * C + 14]
        ids = refs[2 + 2 * C + 14:2 + 2 * C + 18]
        rows = (rows0, rows1)
        gsem = (gsem0, gsem1)
        ssem = (ssem0, ssem1)

        cid = lax.axis_index("c")
        sid = lax.axis_index("s")
        rbase = jnp.where(cid == 0, sid * (NG0 * GR),
                          16 * (NG0 * GR) + sid * (NG1 * GR))

        _zero_scratch(zb, ZR, CW)

        def z(i, _):
            pltpu.sync_copy(zb, acc.at[pl.ds(sid * RPS + i * ZR, ZR)])
            return 0

        lax.fori_loop(0, RPS // ZR, z, 0)
        plsc.subcore_barrier()

        def fire_idx(g, b):
            pltpu.async_copy(gi_hbm.at[pl.ds(rbase + g * GR, GR), :],
                             gi3.at[b], igs[b])
            pltpu.async_copy(dst_hbm.at[pl.ds(rbase + g * GR, GR), :],
                             di3.at[b], ids[b])

        def wait_idx(b):
            pltpu.make_async_copy(gi_hbm.at[pl.ds(0, GR), :], gi3.at[b],
                                  igs[b]).wait()
            pltpu.make_async_copy(dst_hbm.at[pl.ds(0, GR), :], di3.at[b],
                                  ids[b]).wait()

        for c in range(C):
            tab = tables[c]
            out = outs[c]

            def fire_gather(b, bi, tab=tab):
                for k in range(GR):
                    pltpu.async_copy(tab.at[gi3.at[bi, k]],
                                     rows[b].at[pl.ds(k * K, K)], gsem[b])

            def wait_gather(b, bi, tab=tab):
                for k in range(GR):
                    pltpu.make_async_copy(tab.at[gi3.at[bi, k]],
                                          rows[b].at[pl.ds(k * K, K)],
                                          gsem[b]).wait()

            def fire_scatter(b, bi):
                for k in range(GR):
                    pltpu.async_copy(rows[b].at[pl.ds(k * K, K)],
                                     acc.at[di3.at[bi, k]], ssem[b],
                                     add=True)

            def wait_scatter(b, bi):
                for k in range(GR):
                    pltpu.make_async_copy(rows[b].at[pl.ds(k * K, K)],
                                          acc.at[di3.at[bi, k]],
                                          ssem[b]).wait()

            def run(ng):
                fire_idx(0, 0)
                fire_idx(1, 1)
                fire_idx(2, 2)
                wait_idx(0)
                fire_gather(0, 0)

                def outer(go, _):
                    for b4 in (0, 1, 2, 3):
                        g = 4 * go + b4
                        b = b4 % 2
                        nb = 1 - b
                        nbi = (b4 + 1) % 4
                        wait_gather(b, b4)
                        pl.when(g >= 1)(
                            lambda: wait_scatter(nb, (b4 + 3) % 4))
                        pl.when(g + 3 < ng)(
                            lambda: fire_idx(g + 3, (b4 + 3) % 4))
                        @pl.when(g + 1 < ng)
                        def _():
                            wait_idx(nbi)
                            fire_gather(nb, nbi)
                        fire_scatter(b, b4)
                    return 0

                lax.fori_loop(0, ng // 4, outer, 0)
                wait_scatter(1, 3)

            pl.when(cid == 0)(lambda: run(NG0))
            pl.when(cid == 1)(lambda: run(NG1))
            plsc.subcore_barrier()
            pltpu.sync_copy(acc.at[pl.ds(sid * RPS, RPS)],
                            out.at[pl.ds(cid * NPAD + sid * RPS, RPS)])
            if c + 1 < C:
                lax.fori_loop(0, RPS // ZR, z, 0)
                plsc.subcore_barrier()

    return pl.kernel(
        body,
        out_type=tuple(jax.ShapeDtypeStruct((2 * NPAD, CW), jnp.float32)
                       for _ in range(C)),
        mesh=_mesh,
        scratch_types=[
            pltpu.VMEM((4, GR, K), jnp.int32),
            pltpu.VMEM((4, GR, K), jnp.int32),
            pltpu.VMEM((GR * K, CW), jnp.float32),
            pltpu.VMEM((GR * K, CW), jnp.float32),
            pltpu.VMEM((ZR, CW), jnp.float32),
            pltpu.VMEM_SHARED((NPAD, CW), jnp.float32),
            pltpu.SemaphoreType.DMA,
            pltpu.SemaphoreType.DMA,
            pltpu.SemaphoreType.DMA,
            pltpu.SemaphoreType.DMA,
        ] + [pltpu.SemaphoreType.DMA] * 8,
        compiler_params=_sc_params,
    )


_sc_spmm4 = _make_sc_spmm(4)
_sc_spmm10 = _make_sc_spmm(10)


def _tc1_body(x_ref, d0_ref, d1_ref, w0_ref,
              dis_ref, xs0, xs1, xs2, xs3, xw_ref):
    deg = d0_ref[:, 0:1] + d1_ref[:, 0:1]
    dis = jnp.where(deg > 0, lax.rsqrt(jnp.maximum(deg, 1e-12)), 0.0)
    dis_ref[...] = jnp.broadcast_to(dis, (RB, 8))
    xs = x_ref[...] * dis
    for c, ref in enumerate((xs0, xs1, xs2, xs3)):
        ref[...] = xs[:, c * CW:(c + 1) * CW]
    xw_ref[...] = jnp.dot(x_ref[...], w0_ref[...],
                          preferred_element_type=jnp.float32)


def _tc1(x_pad, degp, w0p):
    return pl.pallas_call(
        _tc1_body,
        grid=(NRB,),
        in_specs=[
            pl.BlockSpec((RB, 128), lambda i: (i, 0)),
            pl.BlockSpec((RB, 16), lambda i: (i, 0)),
            pl.BlockSpec((RB, 16), lambda i: (i + NRB, 0)),
            pl.BlockSpec((128, 384), lambda i: (0, 0)),
        ],
        out_specs=[
            pl.BlockSpec((RB, 8), lambda i: (i, 0)),
        ] + [pl.BlockSpec((RB, CW), lambda i: (i, 0)) for _ in range(4)] + [
            pl.BlockSpec((RB, 384), lambda i: (i, 0)),
        ],
        out_shape=[jax.ShapeDtypeStruct((NPAD, 8), jnp.float32)]
        + [jax.ShapeDtypeStruct((NPAD, CW), jnp.float32) for _ in range(4)]
        + [jax.ShapeDtypeStruct((NPAD, 384), jnp.float32)],
    )(x_pad, degp, degp, w0p)


def _tc2_body(*refs):
    (xw_ref, dis_ref, w1_ref, w02_ref, b1_ref), rest = refs[:5], refs[5:]
    s_refs = rest[:8]
    hw_ref = rest[8]
    hs_refs = rest[9:]
    S = jnp.concatenate(
        [s_refs[c][...] + s_refs[c + 4][...] for c in range(4)], axis=1)
    dis = dis_ref[:, 0:1]
    T = S * dis
    h = xw_ref[...] - jnp.dot(T, w1_ref[...],
                              preferred_element_type=jnp.float32) + b1_ref[...]
    h = jnp.maximum(h, 0.0)
    i = pl.program_id(0)
    row = i * RB + lax.broadcasted_iota(jnp.int32, (RB, 1), 0)
    h = jnp.where(row < N, h, 0.0)
    hs = h * dis
    for c, ref in enumerate(hs_refs):
        ref[...] = hs[:, c * CW:(c + 1) * CW]
    hw_ref[...] = jnp.dot(h, w02_ref[...], preferred_element_type=jnp.float32)


def _tc2(xw, s1, dis, w1p, w02p, b1p):
    return pl.pallas_call(
        _tc2_body,
        grid=(NRB,),
        in_specs=[
            pl.BlockSpec((RB, 384), lambda i: (i, 0)),
            pl.BlockSpec((RB, 8), lambda i: (i, 0)),
            pl.BlockSpec((128, 384), lambda i: (0, 0)),
            pl.BlockSpec((384, 512), lambda i: (0, 0)),
            pl.BlockSpec((1, 384), lambda i: (0, 0)),
        ] + [pl.BlockSpec((RB, CW), lambda i: (i, 0)) for _ in range(4)]
        + [pl.BlockSpec((RB, CW), lambda i: (i + NRB, 0)) for _ in range(4)],
        out_specs=[pl.BlockSpec((RB, 512), lambda i: (i, 0))]
        + [pl.BlockSpec((RB, CW), lambda i: (i, 0)) for _ in range(10)],
        out_shape=[jax.ShapeDtypeStruct((NPAD, 512), jnp.float32)]
        + [jax.ShapeDtypeStruct((NPAD, CW), jnp.float32) for _ in range(10)],
    )(xw, dis, w1p, w02p, b1p, *s1, *s1)


def _tc3_body(*refs):
    (hw_ref, dis_ref, w12_ref, b2_ref, wf1_ref, bf1_ref, wf2_ref,
     bf2_ref) = refs[:8]
    s_refs = refs[8:28]
    out_ref = refs[28]
    S = jnp.concatenate(
        [s_refs[c][...] + s_refs[c + 10][...] for c in range(10)], axis=1)
    dis = dis_ref[:, 0:1]
    T = S * dis
    h = hw_ref[...] - jnp.dot(T, w12_ref[...],
                              preferred_element_type=jnp.float32) + b2_ref[...]
    h = jnp.maximum(h, 0.0)
    t = jnp.dot(h, wf1_ref[...], preferred_element_type=jnp.float32)
    t = jnp.maximum(t + bf1_ref[...], 0.0)
    out_ref[...] = jnp.dot(t, wf2_ref[...],
                           preferred_element_type=jnp.float32) + bf2_ref[...]


def _tc3(hw, s2, dis, w12p, b2p, wf1p, bf1p, wf2p, bf2p):
    return pl.pallas_call(
        _tc3_body,
        grid=(NRB,),
        in_specs=[
            pl.BlockSpec((RB, 512), lambda i: (i, 0)),
            pl.BlockSpec((RB, 8), lambda i: (i, 0)),
            pl.BlockSpec((320, 512), lambda i: (0, 0)),
            pl.BlockSpec((1, 512), lambda i: (0, 0)),
            pl.BlockSpec((512, 256), lambda i: (0, 0)),
            pl.BlockSpec((1, 256), lambda i: (0, 0)),
            pl.BlockSpec((256, 8), lambda i: (0, 0)),
            pl.BlockSpec((1, 8), lambda i: (0, 0)),
        ] + [pl.BlockSpec((RB, CW), lambda i: (i, 0)) for _ in range(10)]
        + [pl.BlockSpec((RB, CW), lambda i: (i + NRB, 0)) for _ in range(10)],
        out_specs=pl.BlockSpec((RB, 8), lambda i: (i, 0)),
        out_shape=jax.ShapeDtypeStruct((NPAD, 8), jnp.float32),
    )(hw, dis, w12p, b2p, wf1p, bf1p, wf2p, bf2p, *s2, *s2)


def _pad2(a, r, c):
    return jnp.pad(a, ((0, r - a.shape[0]), (0, c - a.shape[1])))


@jax.jit
def kernel(x, edge_index, W0_1, W1_1, b1, W0_2, W1_2, b2, Wf1, bf1, Wf2, bf2):
    pad_idx = (jnp.arange(E, EPAD, dtype=jnp.int32)) % NPAD
    src = jnp.concatenate([edge_index[0], pad_idx]).reshape(EPAD // K, K)
    dst = jnp.concatenate([edge_index[1], pad_idx]).reshape(EPAD // K, K)
    x_pad = _pad2(x, NPAD, 128)
    w0p = _pad2(W0_1, 128, 384)
    w1p = _pad2(W1_1, 128, 384)
    b1p = _pad2(b1[None, :], 1, 384)
    w02p = _pad2(W0_2, 384, 512)
    w12p = _pad2(W1_2, 320, 512)
    b2p = _pad2(b2[None, :], 1, 512)
    wf1p = _pad2(Wf1, 512, 256)
    bf1p = _pad2(bf1[None, :], 1, 256)
    wf2p = _pad2(Wf2, 256, 8)
    bf2p = _pad2(bf2[None, :], 1, 8)

    degp, gi = _sc_deg(src, dst)
    dis, xs0, xs1, xs2, xs3, xw = _tc1(x_pad, degp, w0p)
    s1 = _sc_spmm4(gi, dst, xs0, xs1, xs2, xs3)
    tc2_out = _tc2(xw, s1, dis, w1p, w02p, b1p)
    hw, hs = tc2_out[0], tc2_out[1:]
    s2 = _sc_spmm10(gi, dst, *hs)
    out = _tc3(hw, s2, dis, w12p, b2p, wf1p, bf1p, wf2p, bf2p)
    return out[:N, 0:1]


# speedup vs baseline: 1.0190x; 1.0190x over previous
import functools

import jax
import jax.numpy as jnp
from jax import lax
from jax.experimental import pallas as pl
from jax.experimental.pallas import tpu as pltpu
from jax.experimental.pallas import tpu_sc as plsc

N = 50000
NPAD = 50176
DUMMY = N
E = 800000
EPAD = 819200
NW = 32
EW = EPAD // NW
K = 128
NB = EW // K
RPS = NPAD // 16
CW = 32
RB = 256
NRB = NPAD // RB

_mesh = plsc.VectorSubcoreMesh(
    core_axis_name="c", subcore_axis_name="s", num_cores=2, num_subcores=16)
_sc_params = pltpu.CompilerParams(use_tc_tiling_on_sc=False)


def _zero_scratch(zb, rows, width):
    z16 = jnp.zeros((16,), jnp.float32)

    def zrow(r, _):
        for j in range(width // 16):
            zb[r, pl.ds(16 * j, 16)] = z16
        return 0

    lax.fori_loop(0, rows, zrow, 0)


SB = 8
NSB = NB // SB
GR = 2
NG = NB // GR
NG0 = 140
NG1 = 60
ZR = 196


def _sc_deg_body(src_hbm, dst_hbm, deg_out, gi_hbm, sib, gib, dib,
                 ones_v, zb, acc):
    cid = lax.axis_index("c")
    sid = lax.axis_index("s")
    wid = sid * 2 + cid
    rbase = wid * (EW // K)

    o16 = jnp.ones((16,), jnp.float32)

    def orow(r, _):
        ones_v[r, :] = o16
        return 0

    lax.fori_loop(0, K, orow, 0)
    _zero_scratch(zb, 64, 16)

    def z(i, _):
        pltpu.sync_copy(zb, acc.at[pl.ds(sid * RPS + i * 64, 64)])
        return 0

    lax.fori_loop(0, RPS // 64, z, 0)
    plsc.subcore_barrier()

    def step(b, _):
        rb = rbase + b * SB
        pltpu.sync_copy(src_hbm.at[pl.ds(rb, SB), :], sib)
        pltpu.sync_copy(dst_hbm.at[pl.ds(rb, SB), :], dib)
        for k in range(SB):
            for j in range(K // 16):
                s = sib[k, pl.ds(16 * j, 16)]
                d = dib[k, pl.ds(16 * j, 16)]
                gib[k, pl.ds(16 * j, 16)] = jnp.where(s == d, DUMMY, s)
        for k in range(SB):
            pltpu.sync_copy(ones_v, acc.at[gib.at[k]], add=True)
        pltpu.sync_copy(gib, gi_hbm.at[pl.ds(rb, SB), :])
        return 0

    lax.fori_loop(0, NSB, step, 0)
    plsc.subcore_barrier()
    pltpu.sync_copy(acc.at[pl.ds(sid * RPS, RPS)],
                    deg_out.at[pl.ds(cid * NPAD + sid * RPS, RPS)])


_sc_deg = pl.kernel(
    _sc_deg_body,
    out_type=(jax.ShapeDtypeStruct((2 * NPAD, 16), jnp.float32),
              jax.ShapeDtypeStruct((EPAD // K, K), jnp.int32)),
    mesh=_mesh,
    scratch_types=[
        pltpu.VMEM((SB, K), jnp.int32),
        pltpu.VMEM((SB, K), jnp.int32),
        pltpu.VMEM((SB, K), jnp.int32),
        pltpu.VMEM((K, 16), jnp.float32),
        pltpu.VMEM((64, 16), jnp.float32),
        pltpu.VMEM_SHARED((NPAD, 16), jnp.float32),
    ],
    compiler_params=_sc_params,
)


def _make_sc_spmm(C):
    def body(*refs):
        gi_hbm, dst_hbm = refs[0], refs[1]
        tables = refs[2:2 + C]
        outs = refs[2 + C:2 + 2 * C]
        (gi3, di3, rows0, rows1, zb, acc,
         gsem0, gsem1, ssem0, ssem1) = refs[2 + 2 * C:2 + 2 * C + 10]
        igs = refs[2 + 2 * C + 10:2 + 2 * C + 14]
        ids = refs[2 + 2 * C + 14:2 + 2 * C + 18]
        rows = (rows0, rows1)
        gsem = (gsem0, gsem1)
        ssem = (ssem0, ssem1)

        cid = lax.axis_index("c")
        sid = lax.axis_index("s")
        rbase = jnp.where(cid == 0, sid * (NG0 * GR),
                          16 * (NG0 * GR) + sid * (NG1 * GR))

        _zero_scratch(zb, ZR, CW)

        def z(i, _):
            pltpu.sync_copy(zb, acc.at[pl.ds(sid * RPS + i * ZR, ZR)])
            return 0

        lax.fori_loop(0, RPS // ZR, z, 0)
        plsc.subcore_barrier()

        def fire_idx(g, b):
            pltpu.async_copy(gi_hbm.at[pl.ds(rbase + g * GR, GR), :],
                             gi3.at[b], igs[b])
            pltpu.async_copy(dst_hbm.at[pl.ds(rbase + g * GR, GR), :],
                             di3.at[b], ids[b])

        def wait_idx(b):
            pltpu.make_async_copy(gi_hbm.at[pl.ds(0, GR), :], gi3.at[b],
                                  igs[b]).wait()
            pltpu.make_async_copy(dst_hbm.at[pl.ds(0, GR), :], di3.at[b],
                                  ids[b]).wait()

        for ci in range(C):
            tab0, out0 = tables[ci], outs[ci]
            tab1, out1 = tables[C - 1 - ci], outs[C - 1 - ci]

            def fire_gather(b, bi, tab):
                for k in range(GR):
                    pltpu.async_copy(tab.at[gi3.at[bi, k]],
                                     rows[b].at[pl.ds(k * K, K)], gsem[b])

            def wait_gather(b, bi, tab):
                for k in range(GR):
                    pltpu.make_async_copy(tab.at[gi3.at[bi, k]],
                                          rows[b].at[pl.ds(k * K, K)],
                                          gsem[b]).wait()

            def fire_scatter(b, bi):
                for k in range(GR):
                    pltpu.async_copy(rows[b].at[pl.ds(k * K, K)],
                                     acc.at[di3.at[bi, k]], ssem[b],
                                     add=True)

            def wait_scatter(b, bi):
                for k in range(GR):
                    pltpu.make_async_copy(rows[b].at[pl.ds(k * K, K)],
                                          acc.at[di3.at[bi, k]],
                                          ssem[b]).wait()

            def run(ng, tab):
                fire_idx(0, 0)
                fire_idx(1, 1)
                fire_idx(2, 2)
                wait_idx(0)
                fire_gather(0, 0, tab)

                def outer(go, _):
                    for b4 in (0, 1, 2, 3):
                        g = 4 * go + b4
                        b = b4 % 2
                        nb = 1 - b
                        nbi = (b4 + 1) % 4
                        wait_gather(b, b4, tab)
                        pl.when(g >= 1)(
                            lambda: wait_scatter(nb, (b4 + 3) % 4))
                        pl.when(g + 3 < ng)(
                            lambda: fire_idx(g + 3, (b4 + 3) % 4))
                        @pl.when(g + 1 < ng)
                        def _():
                            wait_idx(nbi)
                            fire_gather(nb, nbi, tab)
                        fire_scatter(b, b4)
                    return 0

                lax.fori_loop(0, ng // 4, outer, 0)
                wait_scatter(1, 3)

            pl.when(cid == 0)(lambda: run(NG0, tab0))
            pl.when(cid == 1)(lambda: run(NG1, tab1))
            plsc.subcore_barrier()
            pl.when(cid == 0)(lambda: pltpu.sync_copy(
                acc.at[pl.ds(sid * RPS, RPS)],
                out0.at[pl.ds(sid * RPS, RPS)]))
            pl.when(cid == 1)(lambda: pltpu.sync_copy(
                acc.at[pl.ds(sid * RPS, RPS)],
                out1.at[pl.ds(NPAD + sid * RPS, RPS)]))
            if ci + 1 < C:
                lax.fori_loop(0, RPS // ZR, z, 0)
                plsc.subcore_barrier()

    return pl.kernel(
        body,
        out_type=tuple(jax.ShapeDtypeStruct((2 * NPAD, CW), jnp.float32)
                       for _ in range(C)),
        mesh=_mesh,
        scratch_types=[
            pltpu.VMEM((4, GR, K), jnp.int32),
            pltpu.VMEM((4, GR, K), jnp.int32),
            pltpu.VMEM((GR * K, CW), jnp.float32),
            pltpu.VMEM((GR * K, CW), jnp.float32),
            pltpu.VMEM((ZR, CW), jnp.float32),
            pltpu.VMEM_SHARED((NPAD, CW), jnp.float32),
            pltpu.SemaphoreType.DMA,
            pltpu.SemaphoreType.DMA,
            pltpu.SemaphoreType.DMA,
            pltpu.SemaphoreType.DMA,
        ] + [pltpu.SemaphoreType.DMA] * 8,
        compiler_params=_sc_params,
    )


_sc_spmm4 = _make_sc_spmm(4)
_sc_spmm10 = _make_sc_spmm(10)


def _tc1_body(x_ref, d0_ref, d1_ref, w0_ref,
              dis_ref, xs0, xs1, xs2, xs3, xw_ref):
    deg = d0_ref[:, 0:1] + d1_ref[:, 0:1]
    dis = jnp.where(deg > 0, lax.rsqrt(jnp.maximum(deg, 1e-12)), 0.0)
    dis_ref[...] = jnp.broadcast_to(dis, (RB, 8))
    xs = x_ref[...] * dis
    for c, ref in enumerate((xs0, xs1, xs2, xs3)):
        ref[...] = xs[:, c * CW:(c + 1) * CW]
    xw_ref[...] = jnp.dot(x_ref[...], w0_ref[...],
                          preferred_element_type=jnp.float32)


def _tc1(x_pad, degp, w0p):
    return pl.pallas_call(
        _tc1_body,
        grid=(NRB,),
        in_specs=[
            pl.BlockSpec((RB, 128), lambda i: (i, 0)),
            pl.BlockSpec((RB, 16), lambda i: (i, 0)),
            pl.BlockSpec((RB, 16), lambda i: (i + NRB, 0)),
            pl.BlockSpec((128, 384), lambda i: (0, 0)),
        ],
        out_specs=[
            pl.BlockSpec((RB, 8), lambda i: (i, 0)),
        ] + [pl.BlockSpec((RB, CW), lambda i: (i, 0)) for _ in range(4)] + [
            pl.BlockSpec((RB, 384), lambda i: (i, 0)),
        ],
        out_shape=[jax.ShapeDtypeStruct((NPAD, 8), jnp.float32)]
        + [jax.ShapeDtypeStruct((NPAD, CW), jnp.float32) for _ in range(4)]
        + [jax.ShapeDtypeStruct((NPAD, 384), jnp.float32)],
    )(x_pad, degp, degp, w0p)


def _tc2_body(*refs):
    (xw_ref, dis_ref, w1_ref, w02_ref, b1_ref), rest = refs[:5], refs[5:]
    s_refs = rest[:8]
    hw_ref = rest[8]
    hs_refs = rest[9:]
    S = jnp.concatenate(
        [s_refs[c][...] + s_refs[c + 4][...] for c in range(4)], axis=1)
    dis = dis_ref[:, 0:1]
    T = S * dis
    h = xw_ref[...] - jnp.dot(T, w1_ref[...],
                              preferred_element_type=jnp.float32) + b1_ref[...]
    h = jnp.maximum(h, 0.0)
    i = pl.program_id(0)
    row = i * RB + lax.broadcasted_iota(jnp.int32, (RB, 1), 0)
    h = jnp.where(row < N, h, 0.0)
    hs = h * dis
    for c, ref in enumerate(hs_refs):
        ref[...] = hs[:, c * CW:(c + 1) * CW]
    hw_ref[...] = jnp.dot(h, w02_ref[...], preferred_element_type=jnp.float32)


def _tc2(xw, s1, dis, w1p, w02p, b1p):
    return pl.pallas_call(
        _tc2_body,
        grid=(NRB,),
        in_specs=[
            pl.BlockSpec((RB, 384), lambda i: (i, 0)),
            pl.BlockSpec((RB, 8), lambda i: (i, 0)),
            pl.BlockSpec((128, 384), lambda i: (0, 0)),
            pl.BlockSpec((384, 512), lambda i: (0, 0)),
            pl.BlockSpec((1, 384), lambda i: (0, 0)),
        ] + [pl.BlockSpec((RB, CW), lambda i: (i, 0)) for _ in range(4)]
        + [pl.BlockSpec((RB, CW), lambda i: (i + NRB, 0)) for _ in range(4)],
        out_specs=[pl.BlockSpec((RB, 512), lambda i: (i, 0))]
        + [pl.BlockSpec((RB, CW), lambda i: (i, 0)) for _ in range(10)],
        out_shape=[jax.ShapeDtypeStruct((NPAD, 512), jnp.float32)]
        + [jax.ShapeDtypeStruct((NPAD, CW), jnp.float32) for _ in range(10)],
    )(xw, dis, w1p, w02p, b1p, *s1, *s1)


def _tc3_body(*refs):
    (hw_ref, dis_ref, w12_ref, b2_ref, wf1_ref, bf1_ref, wf2_ref,
     bf2_ref) = refs[:8]
    s_refs = refs[8:28]
    out_ref = refs[28]
    S = jnp.concatenate(
        [s_refs[c][...] + s_refs[c + 10][...] for c in range(10)], axis=1)
    dis = dis_ref[:, 0:1]
    T = S * dis
    h = hw_ref[...] - jnp.dot(T, w12_ref[...],
                              preferred_element_type=jnp.float32) + b2_ref[...]
    h = jnp.maximum(h, 0.0)
    t = jnp.dot(h, wf1_ref[...], preferred_element_type=jnp.float32)
    t = jnp.maximum(t + bf1_ref[...], 0.0)
    out_ref[...] = jnp.dot(t, wf2_ref[...],
                           preferred_element_type=jnp.float32) + bf2_ref[...]


def _tc3(hw, s2, dis, w12p, b2p, wf1p, bf1p, wf2p, bf2p):
    return pl.pallas_call(
        _tc3_body,
        grid=(NRB,),
        in_specs=[
            pl.BlockSpec((RB, 512), lambda i: (i, 0)),
            pl.BlockSpec((RB, 8), lambda i: (i, 0)),
            pl.BlockSpec((320, 512), lambda i: (0, 0)),
            pl.BlockSpec((1, 512), lambda i: (0, 0)),
            pl.BlockSpec((512, 256), lambda i: (0, 0)),
            pl.BlockSpec((1, 256), lambda i: (0, 0)),
            pl.BlockSpec((256, 8), lambda i: (0, 0)),
            pl.BlockSpec((1, 8), lambda i: (0, 0)),
        ] + [pl.BlockSpec((RB, CW), lambda i: (i, 0)) for _ in range(10)]
        + [pl.BlockSpec((RB, CW), lambda i: (i + NRB, 0)) for _ in range(10)],
        out_specs=pl.BlockSpec((RB, 8), lambda i: (i, 0)),
        out_shape=jax.ShapeDtypeStruct((NPAD, 8), jnp.float32),
    )(hw, dis, w12p, b2p, wf1p, bf1p, wf2p, bf2p, *s2, *s2)


def _pad2(a, r, c):
    return jnp.pad(a, ((0, r - a.shape[0]), (0, c - a.shape[1])))


@jax.jit
def kernel(x, edge_index, W0_1, W1_1, b1, W0_2, W1_2, b2, Wf1, bf1, Wf2, bf2):
    pad_idx = (jnp.arange(E, EPAD, dtype=jnp.int32)) % NPAD
    src = jnp.concatenate([edge_index[0], pad_idx]).reshape(EPAD // K, K)
    dst = jnp.concatenate([edge_index[1], pad_idx]).reshape(EPAD // K, K)
    x_pad = _pad2(x, NPAD, 128)
    w0p = _pad2(W0_1, 128, 384)
    w1p = _pad2(W1_1, 128, 384)
    b1p = _pad2(b1[None, :], 1, 384)
    w02p = _pad2(W0_2, 384, 512)
    w12p = _pad2(W1_2, 320, 512)
    b2p = _pad2(b2[None, :], 1, 512)
    wf1p = _pad2(Wf1, 512, 256)
    bf1p = _pad2(bf1[None, :], 1, 256)
    wf2p = _pad2(Wf2, 256, 8)
    bf2p = _pad2(bf2[None, :], 1, 8)

    degp, gi = _sc_deg(src, dst)
    dis, xs0, xs1, xs2, xs3, xw = _tc1(x_pad, degp, w0p)
    s1 = _sc_spmm4(gi, dst, xs0, xs1, xs2, xs3)
    tc2_out = _tc2(xw, s1, dis, w1p, w02p, b1p)
    hw, hs = tc2_out[0], tc2_out[1:]
    s2 = _sc_spmm10(gi, dst, *hs)
    out = _tc3(hw, s2, dis, w12p, b2p, wf1p, bf1p, wf2p, bf2p)
    return out[:N, 0:1]


# speedup vs baseline: 1.0653x; 1.0454x over previous
import functools

import jax
import jax.numpy as jnp
from jax import lax
from jax.experimental import pallas as pl
from jax.experimental.pallas import tpu as pltpu
from jax.experimental.pallas import tpu_sc as plsc

N = 50000
NPAD = 50176
DUMMY = N
E = 800000
EPAD = 819200
NW = 32
EW = EPAD // NW
K = 128
NB = EW // K
RPS = NPAD // 16
CW = 32
RB = 256
NRB = NPAD // RB

_mesh = plsc.VectorSubcoreMesh(
    core_axis_name="c", subcore_axis_name="s", num_cores=2, num_subcores=16)
_sc_params = pltpu.CompilerParams(use_tc_tiling_on_sc=False)


def _zero_scratch(zb, rows, width):
    z16 = jnp.zeros((16,), jnp.float32)

    def zrow(r, _):
        for j in range(width // 16):
            zb[r, pl.ds(16 * j, 16)] = z16
        return 0

    lax.fori_loop(0, rows, zrow, 0)


SB = 8
NSB = NB // SB
GR = 2
NG = NB // GR
NG0 = 160
NG1 = 40
ZR = 196


def _sc_deg_body(src_hbm, dst_hbm, deg_out, gi_hbm, sib, gib, dib,
                 ones_v, zb, acc):
    cid = lax.axis_index("c")
    sid = lax.axis_index("s")
    wid = sid * 2 + cid
    rbase = wid * (EW // K)

    o16 = jnp.ones((16,), jnp.float32)

    def orow(r, _):
        ones_v[r, :] = o16
        return 0

    lax.fori_loop(0, K, orow, 0)
    _zero_scratch(zb, 64, 16)

    def z(i, _):
        pltpu.sync_copy(zb, acc.at[pl.ds(sid * RPS + i * 64, 64)])
        return 0

    lax.fori_loop(0, RPS // 64, z, 0)
    plsc.subcore_barrier()

    def step(b, _):
        rb = rbase + b * SB
        pltpu.sync_copy(src_hbm.at[pl.ds(rb, SB), :], sib)
        pltpu.sync_copy(dst_hbm.at[pl.ds(rb, SB), :], dib)
        for k in range(SB):
            for j in range(K // 16):
                s = sib[k, pl.ds(16 * j, 16)]
                d = dib[k, pl.ds(16 * j, 16)]
                gib[k, pl.ds(16 * j, 16)] = jnp.where(s == d, DUMMY, s)
        for k in range(SB):
            pltpu.sync_copy(ones_v, acc.at[gib.at[k]], add=True)
        pltpu.sync_copy(gib, gi_hbm.at[pl.ds(rb, SB), :])
        return 0

    lax.fori_loop(0, NSB, step, 0)
    plsc.subcore_barrier()
    pltpu.sync_copy(acc.at[pl.ds(sid * RPS, RPS)],
                    deg_out.at[pl.ds(cid * NPAD + sid * RPS, RPS)])


_sc_deg = pl.kernel(
    _sc_deg_body,
    out_type=(jax.ShapeDtypeStruct((2 * NPAD, 16), jnp.float32),
              jax.ShapeDtypeStruct((EPAD // K, K), jnp.int32)),
    mesh=_mesh,
    scratch_types=[
        pltpu.VMEM((SB, K), jnp.int32),
        pltpu.VMEM((SB, K), jnp.int32),
        pltpu.VMEM((SB, K), jnp.int32),
        pltpu.VMEM((K, 16), jnp.float32),
        pltpu.VMEM((64, 16), jnp.float32),
        pltpu.VMEM_SHARED((NPAD, 16), jnp.float32),
    ],
    compiler_params=_sc_params,
)


def _make_sc_spmm(C):
    def body(*refs):
        gi_hbm, dst_hbm = refs[0], refs[1]
        tables = refs[2:2 + C]
        outs = refs[2 + C:2 + 2 * C]
        (gi3, di3, rows0, rows1, zb, acc,
         gsem0, gsem1, ssem0, ssem1) = refs[2 + 2 * C:2 + 2 * C + 10]
        igs = refs[2 + 2 * C + 10:2 + 2 * C + 14]
        ids = refs[2 + 2 * C + 14:2 + 2 * C + 18]
        rows = (rows0, rows1)
        gsem = (gsem0, gsem1)
        ssem = (ssem0, ssem1)

        cid = lax.axis_index("c")
        sid = lax.axis_index("s")
        rbase = jnp.where(cid == 0, sid * (NG0 * GR),
                          16 * (NG0 * GR) + sid * (NG1 * GR))

        _zero_scratch(zb, ZR, CW)

        def z(i, _):
            pltpu.sync_copy(zb, acc.at[pl.ds(sid * RPS + i * ZR, ZR)])
            return 0

        lax.fori_loop(0, RPS // ZR, z, 0)
        plsc.subcore_barrier()

        def fire_idx(g, b):
            pltpu.async_copy(gi_hbm.at[pl.ds(rbase + g * GR, GR), :],
                             gi3.at[b], igs[b])
            pltpu.async_copy(dst_hbm.at[pl.ds(rbase + g * GR, GR), :],
                             di3.at[b], ids[b])

        def wait_idx(b):
            pltpu.make_async_copy(gi_hbm.at[pl.ds(0, GR), :], gi3.at[b],
                                  igs[b]).wait()
            pltpu.make_async_copy(dst_hbm.at[pl.ds(0, GR), :], di3.at[b],
                                  ids[b]).wait()

        for ci in range(C):
            tab0, out0 = tables[ci], outs[ci]
            tab1, out1 = tables[C - 1 - ci], outs[C - 1 - ci]

            def fire_gather(b, bi, tab):
                for k in range(GR):
                    pltpu.async_copy(tab.at[gi3.at[bi, k]],
                                     rows[b].at[pl.ds(k * K, K)], gsem[b])

            def wait_gather(b, bi, tab):
                for k in range(GR):
                    pltpu.make_async_copy(tab.at[gi3.at[bi, k]],
                                          rows[b].at[pl.ds(k * K, K)],
                                          gsem[b]).wait()

            def fire_scatter(b, bi):
                for k in range(GR):
                    pltpu.async_copy(rows[b].at[pl.ds(k * K, K)],
                                     acc.at[di3.at[bi, k]], ssem[b],
                                     add=True)

            def wait_scatter(b, bi):
                for k in range(GR):
                    pltpu.make_async_copy(rows[b].at[pl.ds(k * K, K)],
                                          acc.at[di3.at[bi, k]],
                                          ssem[b]).wait()

            def run(ng, tab):
                fire_idx(0, 0)
                fire_idx(1, 1)
                fire_idx(2, 2)
                wait_idx(0)
                fire_gather(0, 0, tab)

                def outer(go, _):
                    for b4 in (0, 1, 2, 3):
                        g = 4 * go + b4
                        b = b4 % 2
                        nb = 1 - b
                        nbi = (b4 + 1) % 4
                        wait_gather(b, b4, tab)
                        pl.when(g >= 1)(
                            lambda: wait_scatter(nb, (b4 + 3) % 4))
                        pl.when(g + 3 < ng)(
                            lambda: fire_idx(g + 3, (b4 + 3) % 4))
                        @pl.when(g + 1 < ng)
                        def _():
                            wait_idx(nbi)
                            fire_gather(nb, nbi, tab)
                        fire_scatter(b, b4)
                    return 0

                lax.fori_loop(0, ng // 4, outer, 0)
                wait_scatter(1, 3)

            pl.when(cid == 0)(lambda: run(NG0, tab0))
            pl.when(cid == 1)(lambda: run(NG1, tab1))
            plsc.subcore_barrier()
            pl.when(cid == 0)(lambda: pltpu.sync_copy(
                acc.at[pl.ds(sid * RPS, RPS)],
                out0.at[pl.ds(sid * RPS, RPS)]))
            pl.when(cid == 1)(lambda: pltpu.sync_copy(
                acc.at[pl.ds(sid * RPS, RPS)],
                out1.at[pl.ds(NPAD + sid * RPS, RPS)]))
            if ci + 1 < C:
                lax.fori_loop(0, RPS // ZR, z, 0)
                plsc.subcore_barrier()

    return pl.kernel(
        body,
        out_type=tuple(jax.ShapeDtypeStruct((2 * NPAD, CW), jnp.float32)
                       for _ in range(C)),
        mesh=_mesh,
        scratch_types=[
            pltpu.VMEM((4, GR, K), jnp.int32),
            pltpu.VMEM((4, GR, K), jnp.int32),
            pltpu.VMEM((GR * K, CW), jnp.float32),
            pltpu.VMEM((GR * K, CW), jnp.float32),
            pltpu.VMEM((ZR, CW), jnp.float32),
            pltpu.VMEM_SHARED((NPAD, CW), jnp.float32),
            pltpu.SemaphoreType.DMA,
            pltpu.SemaphoreType.DMA,
            pltpu.SemaphoreType.DMA,
            pltpu.SemaphoreType.DMA,
        ] + [pltpu.SemaphoreType.DMA] * 8,
        compiler_params=_sc_params,
    )


_sc_spmm4 = _make_sc_spmm(4)
_sc_spmm10 = _make_sc_spmm(10)


def _tc1_body(x_ref, d0_ref, d1_ref, w0_ref,
              dis_ref, xs0, xs1, xs2, xs3, xw_ref):
    deg = d0_ref[:, 0:1] + d1_ref[:, 0:1]
    dis = jnp.where(deg > 0, lax.rsqrt(jnp.maximum(deg, 1e-12)), 0.0)
    dis_ref[...] = jnp.broadcast_to(dis, (RB, 8))
    xs = x_ref[...] * dis
    for c, ref in enumerate((xs0, xs1, xs2, xs3)):
        ref[...] = xs[:, c * CW:(c + 1) * CW]
    xw_ref[...] = jnp.dot(x_ref[...], w0_ref[...],
                          preferred_element_type=jnp.float32)


def _tc1(x_pad, degp, w0p):
    return pl.pallas_call(
        _tc1_body,
        grid=(NRB,),
        in_specs=[
            pl.BlockSpec((RB, 128), lambda i: (i, 0)),
            pl.BlockSpec((RB, 16), lambda i: (i, 0)),
            pl.BlockSpec((RB, 16), lambda i: (i + NRB, 0)),
            pl.BlockSpec((128, 384), lambda i: (0, 0)),
        ],
        out_specs=[
            pl.BlockSpec((RB, 8), lambda i: (i, 0)),
        ] + [pl.BlockSpec((RB, CW), lambda i: (i, 0)) for _ in range(4)] + [
            pl.BlockSpec((RB, 384), lambda i: (i, 0)),
        ],
        out_shape=[jax.ShapeDtypeStruct((NPAD, 8), jnp.float32)]
        + [jax.ShapeDtypeStruct((NPAD, CW), jnp.float32) for _ in range(4)]
        + [jax.ShapeDtypeStruct((NPAD, 384), jnp.float32)],
    )(x_pad, degp, degp, w0p)


def _tc2_body(*refs):
    (xw_ref, dis_ref, w1_ref, w02_ref, b1_ref), rest = refs[:5], refs[5:]
    s_refs = rest[:8]
    hw_ref = rest[8]
    hs_refs = rest[9:]
    S = jnp.concatenate(
        [s_refs[c][...] + s_refs[c + 4][...] for c in range(4)], axis=1)
    dis = dis_ref[:, 0:1]
    T = S * dis
    h = xw_ref[...] - jnp.dot(T, w1_ref[...],
                              preferred_element_type=jnp.float32) + b1_ref[...]
    h = jnp.maximum(h, 0.0)
    i = pl.program_id(0)
    row = i * RB + lax.broadcasted_iota(jnp.int32, (RB, 1), 0)
    h = jnp.where(row < N, h, 0.0)
    hs = h * dis
    for c, ref in enumerate(hs_refs):
        ref[...] = hs[:, c * CW:(c + 1) * CW]
    hw_ref[...] = jnp.dot(h, w02_ref[...], preferred_element_type=jnp.float32)


def _tc2(xw, s1, dis, w1p, w02p, b1p):
    return pl.pallas_call(
        _tc2_body,
        grid=(NRB,),
        in_specs=[
            pl.BlockSpec((RB, 384), lambda i: (i, 0)),
            pl.BlockSpec((RB, 8), lambda i: (i, 0)),
            pl.BlockSpec((128, 384), lambda i: (0, 0)),
            pl.BlockSpec((384, 512), lambda i: (0, 0)),
            pl.BlockSpec((1, 384), lambda i: (0, 0)),
        ] + [pl.BlockSpec((RB, CW), lambda i: (i, 0)) for _ in range(4)]
        + [pl.BlockSpec((RB, CW), lambda i: (i + NRB, 0)) for _ in range(4)],
        out_specs=[pl.BlockSpec((RB, 512), lambda i: (i, 0))]
        + [pl.BlockSpec((RB, CW), lambda i: (i, 0)) for _ in range(10)],
        out_shape=[jax.ShapeDtypeStruct((NPAD, 512), jnp.float32)]
        + [jax.ShapeDtypeStruct((NPAD, CW), jnp.float32) for _ in range(10)],
    )(xw, dis, w1p, w02p, b1p, *s1, *s1)


def _tc3_body(*refs):
    (hw_ref, dis_ref, w12_ref, b2_ref, wf1_ref, bf1_ref, wf2_ref,
     bf2_ref) = refs[:8]
    s_refs = refs[8:28]
    out_ref = refs[28]
    S = jnp.concatenate(
        [s_refs[c][...] + s_refs[c + 10][...] for c in range(10)], axis=1)
    dis = dis_ref[:, 0:1]
    T = S * dis
    h = hw_ref[...] - jnp.dot(T, w12_ref[...],
                              preferred_element_type=jnp.float32) + b2_ref[...]
    h = jnp.maximum(h, 0.0)
    t = jnp.dot(h, wf1_ref[...], preferred_element_type=jnp.float32)
    t = jnp.maximum(t + bf1_ref[...], 0.0)
    out_ref[...] = jnp.dot(t, wf2_ref[...],
                           preferred_element_type=jnp.float32) + bf2_ref[...]


def _tc3(hw, s2, dis, w12p, b2p, wf1p, bf1p, wf2p, bf2p):
    return pl.pallas_call(
        _tc3_body,
        grid=(NRB,),
        in_specs=[
            pl.BlockSpec((RB, 512), lambda i: (i, 0)),
            pl.BlockSpec((RB, 8), lambda i: (i, 0)),
            pl.BlockSpec((320, 512), lambda i: (0, 0)),
            pl.BlockSpec((1, 512), lambda i: (0, 0)),
            pl.BlockSpec((512, 256), lambda i: (0, 0)),
            pl.BlockSpec((1, 256), lambda i: (0, 0)),
            pl.BlockSpec((256, 8), lambda i: (0, 0)),
            pl.BlockSpec((1, 8), lambda i: (0, 0)),
        ] + [pl.BlockSpec((RB, CW), lambda i: (i, 0)) for _ in range(10)]
        + [pl.BlockSpec((RB, CW), lambda i: (i + NRB, 0)) for _ in range(10)],
        out_specs=pl.BlockSpec((RB, 8), lambda i: (i, 0)),
        out_shape=jax.ShapeDtypeStruct((NPAD, 8), jnp.float32),
    )(hw, dis, w12p, b2p, wf1p, bf1p, wf2p, bf2p, *s2, *s2)


def _pad2(a, r, c):
    return jnp.pad(a, ((0, r - a.shape[0]), (0, c - a.shape[1])))


@jax.jit
def kernel(x, edge_index, W0_1, W1_1, b1, W0_2, W1_2, b2, Wf1, bf1, Wf2, bf2):
    pad_idx = (jnp.arange(E, EPAD, dtype=jnp.int32)) % NPAD
    src = jnp.concatenate([edge_index[0], pad_idx]).reshape(EPAD // K, K)
    dst = jnp.concatenate([edge_index[1], pad_idx]).reshape(EPAD // K, K)
    x_pad = _pad2(x, NPAD, 128)
    w0p = _pad2(W0_1, 128, 384)
    w1p = _pad2(W1_1, 128, 384)
    b1p = _pad2(b1[None, :], 1, 384)
    w02p = _pad2(W0_2, 384, 512)
    w12p = _pad2(W1_2, 320, 512)
    b2p = _pad2(b2[None, :], 1, 512)
    wf1p = _pad2(Wf1, 512, 256)
    bf1p = _pad2(bf1[None, :], 1, 256)
    wf2p = _pad2(Wf2, 256, 8)
    bf2p = _pad2(bf2[None, :], 1, 8)

    degp, gi = _sc_deg(src, dst)
    dis, xs0, xs1, xs2, xs3, xw = _tc1(x_pad, degp, w0p)
    s1 = _sc_spmm4(gi, dst, xs0, xs1, xs2, xs3)
    tc2_out = _tc2(xw, s1, dis, w1p, w02p, b1p)
    hw, hs = tc2_out[0], tc2_out[1:]
    s2 = _sc_spmm10(gi, dst, *hs)
    out = _tc3(hw, s2, dis, w12p, b2p, wf1p, bf1p, wf2p, bf2p)
    return out[:N, 0:1]


# speedup vs baseline: 1.0800x; 1.0138x over previous
import functools

import jax
import jax.numpy as jnp
from jax import lax
from jax.experimental import pallas as pl
from jax.experimental.pallas import tpu as pltpu
from jax.experimental.pallas import tpu_sc as plsc

N = 50000
NPAD = 50176
DUMMY = N
E = 800000
EPAD = 819200
NW = 32
EW = EPAD // NW
K = 128
NB = EW // K
RPS = NPAD // 16
CW = 32
RB = 256
NRB = NPAD // RB

_mesh = plsc.VectorSubcoreMesh(
    core_axis_name="c", subcore_axis_name="s", num_cores=2, num_subcores=16)
_sc_params = pltpu.CompilerParams(use_tc_tiling_on_sc=False)


def _zero_scratch(zb, rows, width):
    z16 = jnp.zeros((16,), jnp.float32)

    def zrow(r, _):
        for j in range(width // 16):
            zb[r, pl.ds(16 * j, 16)] = z16
        return 0

    lax.fori_loop(0, rows, zrow, 0)


SB = 8
NSB = NB // SB
GR = 2
NG = NB // GR
NG0 = 168
NG1 = 32
ZR = 196


def _sc_deg_body(src_hbm, dst_hbm, deg_out, gi_hbm, sib, gib, dib,
                 ones_v, zb, acc):
    cid = lax.axis_index("c")
    sid = lax.axis_index("s")
    wid = sid * 2 + cid
    rbase = wid * (EW // K)

    o16 = jnp.ones((16,), jnp.float32)

    def orow(r, _):
        ones_v[r, :] = o16
        return 0

    lax.fori_loop(0, K, orow, 0)
    _zero_scratch(zb, 64, 16)

    def z(i, _):
        pltpu.sync_copy(zb, acc.at[pl.ds(sid * RPS + i * 64, 64)])
        return 0

    lax.fori_loop(0, RPS // 64, z, 0)
    plsc.subcore_barrier()

    def step(b, _):
        rb = rbase + b * SB
        pltpu.sync_copy(src_hbm.at[pl.ds(rb, SB), :], sib)
        pltpu.sync_copy(dst_hbm.at[pl.ds(rb, SB), :], dib)
        for k in range(SB):
            for j in range(K // 16):
                s = sib[k, pl.ds(16 * j, 16)]
                d = dib[k, pl.ds(16 * j, 16)]
                gib[k, pl.ds(16 * j, 16)] = jnp.where(s == d, DUMMY, s)
        for k in range(SB):
            pltpu.sync_copy(ones_v, acc.at[gib.at[k]], add=True)
        pltpu.sync_copy(gib, gi_hbm.at[pl.ds(rb, SB), :])
        return 0

    lax.fori_loop(0, NSB, step, 0)
    plsc.subcore_barrier()
    pltpu.sync_copy(acc.at[pl.ds(sid * RPS, RPS)],
                    deg_out.at[pl.ds(cid * NPAD + sid * RPS, RPS)])


_sc_deg = pl.kernel(
    _sc_deg_body,
    out_type=(jax.ShapeDtypeStruct((2 * NPAD, 16), jnp.float32),
              jax.ShapeDtypeStruct((EPAD // K, K), jnp.int32)),
    mesh=_mesh,
    scratch_types=[
        pltpu.VMEM((SB, K), jnp.int32),
        pltpu.VMEM((SB, K), jnp.int32),
        pltpu.VMEM((SB, K), jnp.int32),
        pltpu.VMEM((K, 16), jnp.float32),
        pltpu.VMEM((64, 16), jnp.float32),
        pltpu.VMEM_SHARED((NPAD, 16), jnp.float32),
    ],
    compiler_params=_sc_params,
)


def _make_sc_spmm(C):
    def body(*refs):
        gi_hbm, dst_hbm = refs[0], refs[1]
        tables = refs[2:2 + C]
        outs = refs[2 + C:2 + 2 * C]
        (gi3, di3, rows0, rows1, zb, acc,
         gsem0, gsem1, ssem0, ssem1) = refs[2 + 2 * C:2 + 2 * C + 10]
        igs = refs[2 + 2 * C + 10:2 + 2 * C + 14]
        ids = refs[2 + 2 * C + 14:2 + 2 * C + 18]
        rows = (rows0, rows1)
        gsem = (gsem0, gsem1)
        ssem = (ssem0, ssem1)

        cid = lax.axis_index("c")
        sid = lax.axis_index("s")
        rbase = jnp.where(cid == 0, sid * (NG0 * GR),
                          16 * (NG0 * GR) + sid * (NG1 * GR))

        _zero_scratch(zb, ZR, CW)

        def z(i, _):
            pltpu.sync_copy(zb, acc.at[pl.ds(sid * RPS + i * ZR, ZR)])
            return 0

        lax.fori_loop(0, RPS // ZR, z, 0)
        plsc.subcore_barrier()

        def fire_idx(g, b):
            pltpu.async_copy(gi_hbm.at[pl.ds(rbase + g * GR, GR), :],
                             gi3.at[b], igs[b])
            pltpu.async_copy(dst_hbm.at[pl.ds(rbase + g * GR, GR), :],
                             di3.at[b], ids[b])

        def wait_idx(b):
            pltpu.make_async_copy(gi_hbm.at[pl.ds(0, GR), :], gi3.at[b],
                                  igs[b]).wait()
            pltpu.make_async_copy(dst_hbm.at[pl.ds(0, GR), :], di3.at[b],
                                  ids[b]).wait()

        for ci in range(C):
            tab0, out0 = tables[ci], outs[ci]
            tab1, out1 = tables[C - 1 - ci], outs[C - 1 - ci]

            def fire_gather(b, bi, tab):
                for k in range(GR):
                    pltpu.async_copy(tab.at[gi3.at[bi, k]],
                                     rows[b].at[pl.ds(k * K, K)], gsem[b])

            def wait_gather(b, bi, tab):
                for k in range(GR):
                    pltpu.make_async_copy(tab.at[gi3.at[bi, k]],
                                          rows[b].at[pl.ds(k * K, K)],
                                          gsem[b]).wait()

            def fire_scatter(b, bi):
                for k in range(GR):
                    pltpu.async_copy(rows[b].at[pl.ds(k * K, K)],
                                     acc.at[di3.at[bi, k]], ssem[b],
                                     add=True)

            def wait_scatter(b, bi):
                for k in range(GR):
                    pltpu.make_async_copy(rows[b].at[pl.ds(k * K, K)],
                                          acc.at[di3.at[bi, k]],
                                          ssem[b]).wait()

            def run(ng, tab):
                fire_idx(0, 0)
                fire_idx(1, 1)
                fire_idx(2, 2)
                wait_idx(0)
                fire_gather(0, 0, tab)

                def outer(go, _):
                    for b4 in (0, 1, 2, 3):
                        g = 4 * go + b4
                        b = b4 % 2
                        nb = 1 - b
                        nbi = (b4 + 1) % 4
                        wait_gather(b, b4, tab)
                        pl.when(g >= 1)(
                            lambda: wait_scatter(nb, (b4 + 3) % 4))
                        pl.when(g + 3 < ng)(
                            lambda: fire_idx(g + 3, (b4 + 3) % 4))
                        @pl.when(g + 1 < ng)
                        def _():
                            wait_idx(nbi)
                            fire_gather(nb, nbi, tab)
                        fire_scatter(b, b4)
                    return 0

                lax.fori_loop(0, ng // 4, outer, 0)
                wait_scatter(1, 3)

            pl.when(cid == 0)(lambda: run(NG0, tab0))
            pl.when(cid == 1)(lambda: run(NG1, tab1))
            plsc.subcore_barrier()
            pl.when(cid == 0)(lambda: pltpu.sync_copy(
                acc.at[pl.ds(sid * RPS, RPS)],
                out0.at[pl.ds(sid * RPS, RPS)]))
            pl.when(cid == 1)(lambda: pltpu.sync_copy(
                acc.at[pl.ds(sid * RPS, RPS)],
                out1.at[pl.ds(NPAD + sid * RPS, RPS)]))
            if ci + 1 < C:
                lax.fori_loop(0, RPS // ZR, z, 0)
                plsc.subcore_barrier()

    return pl.kernel(
        body,
        out_type=tuple(jax.ShapeDtypeStruct((2 * NPAD, CW), jnp.float32)
                       for _ in range(C)),
        mesh=_mesh,
        scratch_types=[
            pltpu.VMEM((4, GR, K), jnp.int32),
            pltpu.VMEM((4, GR, K), jnp.int32),
            pltpu.VMEM((GR * K, CW), jnp.float32),
            pltpu.VMEM((GR * K, CW), jnp.float32),
            pltpu.VMEM((ZR, CW), jnp.float32),
            pltpu.VMEM_SHARED((NPAD, CW), jnp.float32),
            pltpu.SemaphoreType.DMA,
            pltpu.SemaphoreType.DMA,
            pltpu.SemaphoreType.DMA,
            pltpu.SemaphoreType.DMA,
        ] + [pltpu.SemaphoreType.DMA] * 8,
        compiler_params=_sc_params,
    )


_sc_spmm4 = _make_sc_spmm(4)
_sc_spmm10 = _make_sc_spmm(10)


def _tc1_body(x_ref, d0_ref, d1_ref, w0_ref,
              dis_ref, xs0, xs1, xs2, xs3, xw_ref):
    deg = d0_ref[:, 0:1] + d1_ref[:, 0:1]
    dis = jnp.where(deg > 0, lax.rsqrt(jnp.maximum(deg, 1e-12)), 0.0)
    dis_ref[...] = jnp.broadcast_to(dis, (RB, 8))
    xs = x_ref[...] * dis
    for c, ref in enumerate((xs0, xs1, xs2, xs3)):
        ref[...] = xs[:, c * CW:(c + 1) * CW]
    xw_ref[...] = jnp.dot(x_ref[...], w0_ref[...],
                          preferred_element_type=jnp.float32)


def _tc1(x_pad, degp, w0p):
    return pl.pallas_call(
        _tc1_body,
        grid=(NRB,),
        in_specs=[
            pl.BlockSpec((RB, 128), lambda i: (i, 0)),
            pl.BlockSpec((RB, 16), lambda i: (i, 0)),
            pl.BlockSpec((RB, 16), lambda i: (i + NRB, 0)),
            pl.BlockSpec((128, 384), lambda i: (0, 0)),
        ],
        out_specs=[
            pl.BlockSpec((RB, 8), lambda i: (i, 0)),
        ] + [pl.BlockSpec((RB, CW), lambda i: (i, 0)) for _ in range(4)] + [
            pl.BlockSpec((RB, 384), lambda i: (i, 0)),
        ],
        out_shape=[jax.ShapeDtypeStruct((NPAD, 8), jnp.float32)]
        + [jax.ShapeDtypeStruct((NPAD, CW), jnp.float32) for _ in range(4)]
        + [jax.ShapeDtypeStruct((NPAD, 384), jnp.float32)],
    )(x_pad, degp, degp, w0p)


def _tc2_body(*refs):
    (xw_ref, dis_ref, w1_ref, w02_ref, b1_ref), rest = refs[:5], refs[5:]
    s_refs = rest[:8]
    hw_ref = rest[8]
    hs_refs = rest[9:]
    S = jnp.concatenate(
        [s_refs[c][...] + s_refs[c + 4][...] for c in range(4)], axis=1)
    dis = dis_ref[:, 0:1]
    T = S * dis
    h = xw_ref[...] - jnp.dot(T, w1_ref[...],
                              preferred_element_type=jnp.float32) + b1_ref[...]
    h = jnp.maximum(h, 0.0)
    i = pl.program_id(0)
    row = i * RB + lax.broadcasted_iota(jnp.int32, (RB, 1), 0)
    h = jnp.where(row < N, h, 0.0)
    hs = h * dis
    for c, ref in enumerate(hs_refs):
        ref[...] = hs[:, c * CW:(c + 1) * CW]
    hw_ref[...] = jnp.dot(h, w02_ref[...], preferred_element_type=jnp.float32)


def _tc2(xw, s1, dis, w1p, w02p, b1p):
    return pl.pallas_call(
        _tc2_body,
        grid=(NRB,),
        in_specs=[
            pl.BlockSpec((RB, 384), lambda i: (i, 0)),
            pl.BlockSpec((RB, 8), lambda i: (i, 0)),
            pl.BlockSpec((128, 384), lambda i: (0, 0)),
            pl.BlockSpec((384, 512), lambda i: (0, 0)),
            pl.BlockSpec((1, 384), lambda i: (0, 0)),
        ] + [pl.BlockSpec((RB, CW), lambda i: (i, 0)) for _ in range(4)]
        + [pl.BlockSpec((RB, CW), lambda i: (i + NRB, 0)) for _ in range(4)],
        out_specs=[pl.BlockSpec((RB, 512), lambda i: (i, 0))]
        + [pl.BlockSpec((RB, CW), lambda i: (i, 0)) for _ in range(10)],
        out_shape=[jax.ShapeDtypeStruct((NPAD, 512), jnp.float32)]
        + [jax.ShapeDtypeStruct((NPAD, CW), jnp.float32) for _ in range(10)],
    )(xw, dis, w1p, w02p, b1p, *s1, *s1)


def _tc3_body(*refs):
    (hw_ref, dis_ref, w12_ref, b2_ref, wf1_ref, bf1_ref, wf2_ref,
     bf2_ref) = refs[:8]
    s_refs = refs[8:28]
    out_ref = refs[28]
    S = jnp.concatenate(
        [s_refs[c][...] + s_refs[c + 10][...] for c in range(10)], axis=1)
    dis = dis_ref[:, 0:1]
    T = S * dis
    h = hw_ref[...] - jnp.dot(T, w12_ref[...],
                              preferred_element_type=jnp.float32) + b2_ref[...]
    h = jnp.maximum(h, 0.0)
    t = jnp.dot(h, wf1_ref[...], preferred_element_type=jnp.float32)
    t = jnp.maximum(t + bf1_ref[...], 0.0)
    out_ref[...] = jnp.dot(t, wf2_ref[...],
                           preferred_element_type=jnp.float32) + bf2_ref[...]


def _tc3(hw, s2, dis, w12p, b2p, wf1p, bf1p, wf2p, bf2p):
    return pl.pallas_call(
        _tc3_body,
        grid=(NRB,),
        in_specs=[
            pl.BlockSpec((RB, 512), lambda i: (i, 0)),
            pl.BlockSpec((RB, 8), lambda i: (i, 0)),
            pl.BlockSpec((320, 512), lambda i: (0, 0)),
            pl.BlockSpec((1, 512), lambda i: (0, 0)),
            pl.BlockSpec((512, 256), lambda i: (0, 0)),
            pl.BlockSpec((1, 256), lambda i: (0, 0)),
            pl.BlockSpec((256, 8), lambda i: (0, 0)),
            pl.BlockSpec((1, 8), lambda i: (0, 0)),
        ] + [pl.BlockSpec((RB, CW), lambda i: (i, 0)) for _ in range(10)]
        + [pl.BlockSpec((RB, CW), lambda i: (i + NRB, 0)) for _ in range(10)],
        out_specs=pl.BlockSpec((RB, 8), lambda i: (i, 0)),
        out_shape=jax.ShapeDtypeStruct((NPAD, 8), jnp.float32),
    )(hw, dis, w12p, b2p, wf1p, bf1p, wf2p, bf2p, *s2, *s2)


def _pad2(a, r, c):
    return jnp.pad(a, ((0, r - a.shape[0]), (0, c - a.shape[1])))


@jax.jit
def kernel(x, edge_index, W0_1, W1_1, b1, W0_2, W1_2, b2, Wf1, bf1, Wf2, bf2):
    pad_idx = (jnp.arange(E, EPAD, dtype=jnp.int32)) % NPAD
    src = jnp.concatenate([edge_index[0], pad_idx]).reshape(EPAD // K, K)
    dst = jnp.concatenate([edge_index[1], pad_idx]).reshape(EPAD // K, K)
    x_pad = _pad2(x, NPAD, 128)
    w0p = _pad2(W0_1, 128, 384)
    w1p = _pad2(W1_1, 128, 384)
    b1p = _pad2(b1[None, :], 1, 384)
    w02p = _pad2(W0_2, 384, 512)
    w12p = _pad2(W1_2, 320, 512)
    b2p = _pad2(b2[None, :], 1, 512)
    wf1p = _pad2(Wf1, 512, 256)
    bf1p = _pad2(bf1[None, :], 1, 256)
    wf2p = _pad2(Wf2, 256, 8)
    bf2p = _pad2(bf2[None, :], 1, 8)

    degp, gi = _sc_deg(src, dst)
    dis, xs0, xs1, xs2, xs3, xw = _tc1(x_pad, degp, w0p)
    s1 = _sc_spmm4(gi, dst, xs0, xs1, xs2, xs3)
    tc2_out = _tc2(xw, s1, dis, w1p, w02p, b1p)
    hw, hs = tc2_out[0], tc2_out[1:]
    s2 = _sc_spmm10(gi, dst, *hs)
    out = _tc3(hw, s2, dis, w12p, b2p, wf1p, bf1p, wf2p, bf2p)
    return out[:N, 0:1]
